# baseline (speedup 1.0000x reference)
import functools

import jax
import jax.numpy as jnp
from jax import lax
from jax.experimental import pallas as pl
from jax.experimental.pallas import tpu as pltpu
from jax.experimental.pallas import tpu_sc as plsc

VOCAB = 100000
EMB = 64
BATCH = 4096
SEQ = 50
LANES = 16
NC, NS = 2, 16
NW = NC * NS
B_PER_W = BATCH // NW
NBUF = 2


def _sc_lookup(table_pad, seq):
    mesh = plsc.VectorSubcoreMesh(
        core_axis_name="c", subcore_axis_name="s",
        num_cores=NC, num_subcores=NS)

    @functools.partial(
        pl.kernel,
        out_type=jax.ShapeDtypeStruct((SEQ, EMB, BATCH), jnp.float32),
        mesh=mesh,
        scratch_types=[
            pltpu.VMEM((B_PER_W, SEQ), jnp.int32),
            pltpu.VMEM((SEQ, B_PER_W), jnp.int32),
            [pltpu.VMEM((B_PER_W, 2 * EMB), jnp.float32)
             for _ in range(NBUF)],
            [pltpu.VMEM((EMB, B_PER_W), jnp.float32) for _ in range(NBUF)],
            [pltpu.SemaphoreType.DMA for _ in range(NBUF)],
            [pltpu.SemaphoreType.DMA for _ in range(NBUF)],
        ],
        compiler_params=pltpu.CompilerParams(
            use_tc_tiling_on_sc=True, needs_layout_passes=False),
    )
    def k(table_hbm, seq_hbm, out_hbm, idx_v, idx_t, rows, trans,
          gsems, ssems):
        wid = lax.axis_index("s") * NC + lax.axis_index("c")
        b0 = wid * B_PER_W
        pltpu.sync_copy(seq_hbm.at[pl.ds(b0, B_PER_W)], idx_v)

        lane = lax.iota(jnp.int32, LANES)

        def idx_transpose(s, carry):
            col = jnp.full((LANES,), s, jnp.int32)
            for g in range(B_PER_W // LANES):
                vals = plsc.load_gather(idx_v, [lane + g * LANES, col])
                idx_t[s, pl.ds(g * LANES, LANES)] = vals
            return carry

        lax.fori_loop(0, SEQ, idx_transpose, 0)

        for i in range(NBUF):
            pltpu.async_copy(table_hbm.at[idx_t.at[i]], rows[i], gsems[i])

        def outer(g, carry):
            for i in range(NBUF):
                s = g * NBUF + i
                pltpu.make_async_copy(
                    table_hbm.at[idx_t.at[s]], rows[i], gsems[i]).wait()

                @pl.when(g > 0)
                def _():
                    pltpu.make_async_copy(
                        trans[i],
                        out_hbm.at[s - NBUF, :, pl.ds(b0, B_PER_W)],
                        ssems[i]).wait()

                for h in range(B_PER_W // LANES):
                    row_h = lane + h * LANES
                    for e in range(EMB):
                        vals = plsc.load_gather(
                            rows[i], [row_h, jnp.full((LANES,), e,
                                                      jnp.int32)])
                        trans[i][e, pl.ds(h * LANES, LANES)] = vals
                pltpu.async_copy(
                    trans[i], out_hbm.at[s, :, pl.ds(b0, B_PER_W)], ssems[i])
                nxt = s + NBUF

                @pl.when(nxt < SEQ)
                def _():
                    pltpu.async_copy(
                        table_hbm.at[idx_t.at[nxt]], rows[i], gsems[i])
            return carry

        lax.fori_loop(0, SEQ // NBUF, outer, 0)
        for i in range(NBUF):
            pltpu.make_async_copy(
                trans[i],
                out_hbm.at[SEQ - NBUF + i, :, pl.ds(b0, B_PER_W)],
                ssems[i]).wait()

    return k(table_pad, seq)


def kernel(seq, W):
    table_pad = jnp.pad(W, ((0, 0), (0, 2 * EMB - W.shape[1])))
    out_t = _sc_lookup(table_pad, seq.astype(jnp.int32))
    return out_t.transpose(2, 0, 1)


# speedup vs baseline: 1.3450x; 1.3450x over previous
import functools

import jax
import jax.numpy as jnp
from jax import lax
from jax.experimental import pallas as pl
from jax.experimental.pallas import tpu as pltpu
from jax.experimental.pallas import tpu_sc as plsc

VOCAB = 100000
EMB = 64
BATCH = 4096
SEQ = 50
LANES = 16
NC, NS = 2, 16
NW = NC * NS
B_PER_W = BATCH // NW
NBUF = 2


def _sc_lookup(table_pad, seq):
    mesh = plsc.VectorSubcoreMesh(
        core_axis_name="c", subcore_axis_name="s",
        num_cores=NC, num_subcores=NS)

    @functools.partial(
        pl.kernel,
        out_type=jax.ShapeDtypeStruct((SEQ, EMB, BATCH), jnp.float32),
        mesh=mesh,
        scratch_types=[
            pltpu.VMEM((B_PER_W, SEQ), jnp.int32),
            pltpu.VMEM((SEQ, B_PER_W), jnp.int32),
            [pltpu.VMEM((B_PER_W, 2 * EMB), jnp.float32)
             for _ in range(NBUF)],
            [pltpu.VMEM((EMB, B_PER_W), jnp.float32) for _ in range(NBUF)],
            [pltpu.SemaphoreType.DMA for _ in range(NBUF)],
            [pltpu.SemaphoreType.DMA for _ in range(NBUF)],
        ],
        compiler_params=pltpu.CompilerParams(
            use_tc_tiling_on_sc=True, needs_layout_passes=False),
    )
    def k(table_hbm, seq_hbm, out_hbm, idx_v, idx_t, rows, trans,
          gsems, ssems):
        wid = lax.axis_index("s") * NC + lax.axis_index("c")
        b0 = wid * B_PER_W
        pltpu.sync_copy(seq_hbm.at[pl.ds(b0, B_PER_W)], idx_v)

        lane = lax.iota(jnp.int32, LANES)

        def idx_transpose(s, carry):
            col = jnp.full((LANES,), s, jnp.int32)
            for g in range(B_PER_W // LANES):
                vals = plsc.load_gather(idx_v, [lane + g * LANES, col])
                idx_t[s, pl.ds(g * LANES, LANES)] = vals
            return carry

        lax.fori_loop(0, SEQ, idx_transpose, 0)

        for i in range(NBUF):
            pltpu.async_copy(table_hbm.at[idx_t.at[i]], rows[i], gsems[i])

        def outer(g, carry):
            for i in range(NBUF):
                s = g * NBUF + i
                pltpu.make_async_copy(
                    table_hbm.at[idx_t.at[s]], rows[i], gsems[i]).wait()

                @pl.when(g > 0)
                def _():
                    pltpu.make_async_copy(
                        trans[i],
                        out_hbm.at[s - NBUF, :, pl.ds(b0, B_PER_W)],
                        ssems[i]).wait()

                def emb_transpose(r, carry2):
                    rcol = jnp.full((LANES,), r, jnp.int32)
                    for l in range(EMB // LANES):
                        vals = rows[i][r, pl.ds(l * LANES, LANES)]
                        plsc.store_scatter(
                            trans[i], [lane + l * LANES, rcol], vals)
                    return carry2

                lax.fori_loop(0, B_PER_W, emb_transpose, 0)
                pltpu.async_copy(
                    trans[i], out_hbm.at[s, :, pl.ds(b0, B_PER_W)], ssems[i])
                nxt = s + NBUF

                @pl.when(nxt < SEQ)
                def _():
                    pltpu.async_copy(
                        table_hbm.at[idx_t.at[nxt]], rows[i], gsems[i])
            return carry

        lax.fori_loop(0, SEQ // NBUF, outer, 0)
        for i in range(NBUF):
            pltpu.make_async_copy(
                trans[i],
                out_hbm.at[SEQ - NBUF + i, :, pl.ds(b0, B_PER_W)],
                ssems[i]).wait()

    return k(table_pad, seq)


def kernel(seq, W):
    table_pad = jnp.pad(W, ((0, 0), (0, 2 * EMB - W.shape[1])))
    out_t = _sc_lookup(table_pad, seq.astype(jnp.int32))
    return out_t.transpose(2, 0, 1)


# speedup vs baseline: 1.9533x; 1.4523x over previous
import functools

import jax
import jax.numpy as jnp
from jax import lax
from jax.experimental import pallas as pl
from jax.experimental.pallas import tpu as pltpu
from jax.experimental.pallas import tpu_sc as plsc

VOCAB = 100000
EMB = 64
BATCH = 4096
SEQ = 50
LANES = 16
NC, NS = 2, 16
NW = NC * NS
B_PER_W = BATCH // NW
NBUF = 4


def _sc_lookup(table_pad, seq):
    mesh = plsc.VectorSubcoreMesh(
        core_axis_name="c", subcore_axis_name="s",
        num_cores=NC, num_subcores=NS)

    @functools.partial(
        pl.kernel,
        out_type=jax.ShapeDtypeStruct((BATCH, SEQ, EMB), jnp.float32),
        mesh=mesh,
        scratch_types=[
            pltpu.VMEM((B_PER_W, SEQ), jnp.int32),
            [pltpu.VMEM((SEQ, 2 * EMB), jnp.float32) for _ in range(NBUF)],
            [pltpu.VMEM((SEQ, EMB), jnp.float32) for _ in range(NBUF)],
            [pltpu.SemaphoreType.DMA for _ in range(NBUF)],
            [pltpu.SemaphoreType.DMA for _ in range(NBUF)],
        ],
        compiler_params=pltpu.CompilerParams(use_tc_tiling_on_sc=True),
    )
    def k(table_hbm, seq_hbm, out_hbm, idx_v, rows, stages, gsems, ssems):
        wid = lax.axis_index("s") * NC + lax.axis_index("c")
        b0 = wid * B_PER_W
        pltpu.sync_copy(seq_hbm.at[pl.ds(b0, B_PER_W)], idx_v)

        for b in range(NBUF):
            pltpu.async_copy(table_hbm.at[idx_v.at[b]], rows[b], gsems[b])

        def repack(rbuf, sbuf):
            def per_row(r, carry):
                for l in range(EMB // LANES):
                    sbuf[r, pl.ds(l * LANES, LANES)] = (
                        rbuf[r, pl.ds(l * LANES, LANES)])
                return carry
            lax.fori_loop(0, SEQ, per_row, 0)

        def outer(g, carry):
            for b in range(NBUF):
                c = g * NBUF + b
                pltpu.make_async_copy(
                    table_hbm.at[idx_v.at[c]], rows[b], gsems[b]).wait()

                @pl.when(g > 0)
                def _():
                    pltpu.make_async_copy(
                        stages[b], out_hbm.at[b0 + c - NBUF], ssems[b]).wait()

                repack(rows[b], stages[b])
                pltpu.async_copy(stages[b], out_hbm.at[b0 + c], ssems[b])
                nxt = c + NBUF

                @pl.when(nxt < B_PER_W)
                def _():
                    pltpu.async_copy(
                        table_hbm.at[idx_v.at[nxt]], rows[b], gsems[b])
            return carry

        lax.fori_loop(0, B_PER_W // NBUF, outer, 0)
        for b in range(NBUF):
            pltpu.make_async_copy(
                stages[b], out_hbm.at[b0 + B_PER_W - NBUF + b], ssems[b]).wait()

    return k(table_pad, seq)


def kernel(seq, W):
    table_pad = jnp.pad(W, ((0, 0), (0, 2 * EMB - W.shape[1])))
    return _sc_lookup(table_pad, seq.astype(jnp.int32))
